# baseline (speedup 1.0000x reference)
import functools

import jax
import jax.numpy as jnp
from jax import lax
from jax.experimental import pallas as pl
from jax.experimental.pallas import tpu as pltpu
from jax.experimental.pallas import tpu_sc as plsc

NC = 2
NS = 16
NW = NC * NS

N = 10000
NPAD = 10240
ROWS_T = NPAD // NS
EB = 128
LB = 128

_mesh = plsc.VectorSubcoreMesh(
    core_axis_name="c", subcore_axis_name="s", num_cores=NC, num_subcores=NS)


def _make_deg(EPAD):
    ti = EPAD // NW // EB

    @functools.partial(
        pl.kernel,
        out_type=jax.ShapeDtypeStruct((NC, NPAD, 128), jnp.float32),
        mesh=_mesh,
        scratch_types=[
            pltpu.VMEM((ti, EB), jnp.int32),
            pltpu.VMEM((EB, 128), jnp.float32),
            pltpu.SemaphoreType.DMA,
            pltpu.VMEM_SHARED((NPAD, 128), jnp.float32),
        ],
    )
    def deg_k(dst2_hbm, zeros_hbm, ones_hbm, out_hbm, dall, ones_v, ssem,
              acc):
        c = lax.axis_index("c")
        s = lax.axis_index("s")
        w = s * NC + c
        pltpu.sync_copy(zeros_hbm, acc.at[pl.ds(s * ROWS_T, ROWS_T)])
        pltpu.sync_copy(ones_hbm, ones_v)
        pltpu.sync_copy(dst2_hbm.at[pl.ds(pl.multiple_of(w * ti, 8), ti)], dall)
        plsc.subcore_barrier()

        pltpu.async_copy(ones_v, acc.at[dall.at[0]], ssem, add=True)

        def step(j, carry):
            pltpu.async_copy(ones_v, acc.at[dall.at[j]], ssem, add=True)
            pltpu.make_async_copy(ones_v, acc.at[dall.at[0]], ssem).wait()
            return carry

        lax.fori_loop(1, ti, step, 0)
        pltpu.make_async_copy(ones_v, acc.at[dall.at[0]], ssem).wait()
        plsc.subcore_barrier()
        pltpu.sync_copy(acc.at[pl.ds(s * ROWS_T, ROWS_T)],
                        out_hbm.at[c, pl.ds(s * ROWS_T, ROWS_T)])

    return deg_k


def _make_prop(EPAD, split_cols):
    CH = 16
    ti = EPAD // (NS if split_cols else NW) // EB
    assert ti % CH == 0

    scratch = [
        pltpu.VMEM((CH, EB), jnp.int32),
        pltpu.VMEM((CH, EB), jnp.int32),
        pltpu.VMEM((EB, 128), jnp.float32),
        pltpu.VMEM((EB, 128), jnp.float32),
        pltpu.SemaphoreType.DMA,
        pltpu.SemaphoreType.DMA,
        pltpu.VMEM_SHARED((NPAD, 128), jnp.float32),
    ]

    def body(tabs, src2_hbm, dst2_hbm, zeros_hbm, out_hbm,
             sidx, didx, rows0, rows1, gsem, ssem, acc):
        c = lax.axis_index("c")
        s = lax.axis_index("s")
        pltpu.sync_copy(zeros_hbm, acc.at[pl.ds(s * ROWS_T, ROWS_T)])
        base_row = (s if split_cols else s * NC + c) * ti
        plsc.subcore_barrier()

        def run(tab):
            def g_issue(j, buf):
                pltpu.async_copy(tab.at[sidx.at[j]], buf, gsem)

            def g_wait(buf):
                pltpu.make_async_copy(tab.at[sidx.at[0]], buf, gsem).wait()

            def s_issue(j, buf):
                pltpu.async_copy(buf, acc.at[didx.at[j]], ssem, add=True)

            def s_wait():
                pltpu.make_async_copy(
                    rows0, acc.at[didx.at[0]], ssem).wait()

            def chunk(cc, carry):
                row0 = pl.multiple_of(base_row + cc * CH, 8)
                pltpu.sync_copy(src2_hbm.at[pl.ds(row0, CH)], sidx)
                pltpu.sync_copy(dst2_hbm.at[pl.ds(row0, CH)], didx)
                g_issue(0, rows0)
                g_wait(rows0)
                g_issue(1, rows1)
                s_issue(0, rows0)

                def step2(jj, c2):
                    k = 2 * jj
                    g_wait(rows1)
                    s_wait()
                    g_issue(k + 2, rows0)
                    s_issue(k + 1, rows1)
                    g_wait(rows0)
                    s_wait()
                    g_issue(k + 3, rows1)
                    s_issue(k + 2, rows0)
                    return c2

                lax.fori_loop(0, (CH - 2) // 2, step2, 0)
                g_wait(rows1)
                s_wait()
                s_issue(CH - 1, rows1)
                s_wait()
                return carry

            lax.fori_loop(0, ti // CH, chunk, 0)

        if split_cols:
            @pl.when(c == 0)
            def _():
                run(tabs[0])

            @pl.when(c == 1)
            def _():
                run(tabs[1])
        else:
            run(tabs[0])
        plsc.subcore_barrier()
        pltpu.sync_copy(acc.at[pl.ds(s * ROWS_T, ROWS_T)],
                        out_hbm.at[c, pl.ds(s * ROWS_T, ROWS_T)])

    kw = dict(
        out_type=jax.ShapeDtypeStruct((NC, NPAD, 128), jnp.float32),
        mesh=_mesh,
        scratch_types=scratch,
    )
    if split_cols:
        @functools.partial(pl.kernel, **kw)
        def prop_k(ta, tb, src_hbm, dst_hbm, zeros_hbm, out_hbm, *scr):
            body((ta, tb), src_hbm, dst_hbm, zeros_hbm, out_hbm, *scr)
    else:
        @functools.partial(pl.kernel, **kw)
        def prop_k(ta, src_hbm, dst_hbm, zeros_hbm, out_hbm, *scr):
            body((ta,), src_hbm, dst_hbm, zeros_hbm, out_hbm, *scr)
    return prop_k


def _make_link(EP):
    ti = EP // NW // LB
    assert ti % 2 == 0
    per_w = EP // NW

    @functools.partial(
        pl.kernel,
        out_type=jax.ShapeDtypeStruct((EP, 16), jnp.float32),
        mesh=_mesh,
        scratch_types=[
            pltpu.VMEM((per_w,), jnp.int32),
            pltpu.VMEM((per_w,), jnp.int32),
            pltpu.VMEM((LB, 128), jnp.float32),
            pltpu.VMEM((LB, 128), jnp.float32),
            pltpu.VMEM((LB, 128), jnp.float32),
            pltpu.VMEM((LB, 128), jnp.float32),
            pltpu.VMEM((LB, 16), jnp.float32),
            pltpu.SemaphoreType.DMA,
        ],
    )
    def link_k(tab_hbm, ia_hbm, ib_hbm, out_hbm,
               iav, ibv, ra0, rb0, ra1, rb1, tbuf, gsem):
        c = lax.axis_index("c")
        s = lax.axis_index("s")
        w = s * NC + c
        base = pl.multiple_of(w * per_w, 8)
        pltpu.sync_copy(ia_hbm.at[pl.ds(base, per_w)], iav)
        pltpu.sync_copy(ib_hbm.at[pl.ds(base, per_w)], ibv)

        def g_issue(j, bufa, bufb):
            off = pl.multiple_of(j * LB, 8)
            pltpu.async_copy(tab_hbm.at[iav.at[pl.ds(off, LB)]], bufa, gsem)
            pltpu.async_copy(tab_hbm.at[ibv.at[pl.ds(off, LB)]], bufb, gsem)

        def g_wait(bufa, bufb):
            pltpu.make_async_copy(
                tab_hbm.at[iav.at[pl.ds(0, LB)]], bufa, gsem).wait()
            pltpu.make_async_copy(
                tab_hbm.at[ibv.at[pl.ds(0, LB)]], bufb, gsem).wait()

        def compute(j, ra, rb):
            @plsc.parallel_loop(0, LB, step=1, unroll=4)
            def pair(p):
                t = ra[p, pl.ds(0, 16)] * rb[p, pl.ds(0, 16)]
                for k in range(1, 8):
                    t = t + (ra[p, pl.ds(16 * k, 16)]
                             * rb[p, pl.ds(16 * k, 16)])
                tbuf[p, :] = t

            pltpu.sync_copy(
                tbuf, out_hbm.at[pl.ds(base + j * LB, LB)])

        g_issue(0, ra0, rb0)

        def step2(jj, carry):
            j0 = 2 * jj
            g_wait(ra0, rb0)
            g_issue(j0 + 1, ra1, rb1)
            compute(j0, ra0, rb0)
            g_wait(ra1, rb1)
            g_issue(lax.min(j0 + 2, ti - 1), ra0, rb0)
            compute(j0 + 1, ra1, rb1)
            return carry

        lax.fori_loop(0, ti // 2, step2, 0)
        g_wait(ra0, rb0)

    return link_k


_BN = 1000


def _tc_call(body, grid, in_specs, out_specs, out_shapes):
    return pl.pallas_call(
        body,
        grid=grid,
        in_specs=in_specs,
        out_specs=out_specs,
        out_shape=out_shapes,
    )


def _prep_body(x_ref, degp_ref, xhat_ref, dinv_ref):
    d = degp_ref[0] + degp_ref[1] + 1.0
    dv = lax.rsqrt(d)
    dinv_ref[...] = dv[:, :16]
    xhat_ref[...] = x_ref[...] * dv[:, 0:1]


def _k2_body(p_ref, xhat_ref, dinv_ref, w_ref, b_ref, ha_ref, hb_ref):
    dv = dinv_ref[:, 0:1]
    u = (p_ref[0] + p_ref[1] + xhat_ref[...]) * dv
    z = jnp.maximum(
        jnp.dot(u, w_ref[...], preferred_element_type=jnp.float32)
        + b_ref[...], 0.0)
    h = z * dv
    ha_ref[...] = h[:, :128]
    hb_ref[...] = h[:, 128:]


def _k3_body(p_ref, ha_ref, hb_ref, dinv_ref, w2_ref, b2_ref, w3_ref,
             g_ref):
    dv = dinv_ref[:, 0:1]
    ua = (p_ref[0] + ha_ref[...]) * dv
    ub = (p_ref[1] + hb_ref[...]) * dv
    u = jnp.concatenate([ua, ub], axis=1)
    z2 = jnp.maximum(
        jnp.dot(u, w2_ref[...], preferred_element_type=jnp.float32)
        + b2_ref[...], 0.0)
    g = jnp.dot(z2, w3_ref[...], preferred_element_type=jnp.float32)
    g_ref[...] = g * dv


def _k4_body(p_ref, g_ref, dinv_ref, b3_ref, wc_ref, bc_ref,
             z_ref, logit_ref):
    dv = dinv_ref[:, 0:1]
    z3 = jnp.maximum(
        (p_ref[0] + p_ref[1] + g_ref[...]) * dv + b3_ref[...], 0.0)
    z_ref[...] = z3
    logit_ref[...] = (
        jnp.dot(z3, wc_ref[...], preferred_element_type=jnp.float32)
        + bc_ref[...])


def _k5_body(t_ref, g_ref, out_ref):
    s = jnp.dot(t_ref[...], g_ref[...], preferred_element_type=jnp.float32)
    out_ref[...] = 1.0 / (1.0 + jnp.exp(-s))


def kernel(x, edge_index, pos_edge, neg_edge, W1, b1, W2, b2, W3, b3, Wc, bc):
    E = edge_index.shape[1]
    src = edge_index[0].astype(jnp.int32)
    dst = edge_index[1].astype(jnp.int32)

    EPAD = -(-E // (NW * 2 * EB)) * (NW * 2 * EB)
    pad_src = (jnp.arange(EPAD - E) % N).astype(jnp.int32)
    pad_dst = (N + jnp.arange(EPAD - E) % (NPAD - N)).astype(jnp.int32)
    src2 = jnp.concatenate([src, pad_src]).reshape(-1, EB)
    dst2 = jnp.concatenate([dst, pad_dst]).reshape(-1, EB)

    zeros128 = jnp.zeros((ROWS_T, 128), jnp.float32)
    ones128 = jnp.ones((EB, 128), jnp.float32)

    degp = _make_deg(EPAD)(dst2, zeros128, ones128)

    grid = (N // _BN,)
    bspec = pl.BlockSpec

    xhat, dinv = _tc_call(
        _prep_body, grid,
        [bspec((_BN, 128), lambda i: (i, 0)),
         bspec((NC, _BN, 128), lambda i: (0, i, 0))],
        [bspec((_BN, 128), lambda i: (i, 0)),
         bspec((_BN, 16), lambda i: (i, 0))],
        [jax.ShapeDtypeStruct((N, 128), jnp.float32),
         jax.ShapeDtypeStruct((N, 16), jnp.float32)],
    )(x, degp)

    prop128 = _make_prop(EPAD, split_cols=False)
    prop256 = _make_prop(EPAD, split_cols=True)

    p1 = prop128(xhat, src2, dst2, zeros128)

    b1r = b1.reshape(1, -1)
    ha, hb = _tc_call(
        _k2_body, grid,
        [bspec((NC, _BN, 128), lambda i: (0, i, 0)),
         bspec((_BN, 128), lambda i: (i, 0)),
         bspec((_BN, 16), lambda i: (i, 0)),
         bspec((128, 256), lambda i: (0, 0)),
         bspec((1, 256), lambda i: (0, 0))],
        [bspec((_BN, 128), lambda i: (i, 0)),
         bspec((_BN, 128), lambda i: (i, 0))],
        [jax.ShapeDtypeStruct((N, 128), jnp.float32),
         jax.ShapeDtypeStruct((N, 128), jnp.float32)],
    )(p1, xhat, dinv, W1, b1r)

    p2 = prop256(ha, hb, src2, dst2, zeros128)

    b2r = b2.reshape(1, -1)
    ghat = _tc_call(
        _k3_body, grid,
        [bspec((NC, _BN, 128), lambda i: (0, i, 0)),
         bspec((_BN, 128), lambda i: (i, 0)),
         bspec((_BN, 128), lambda i: (i, 0)),
         bspec((_BN, 16), lambda i: (i, 0)),
         bspec((256, 256), lambda i: (0, 0)),
         bspec((1, 256), lambda i: (0, 0)),
         bspec((256, 128), lambda i: (0, 0))],
        bspec((_BN, 128), lambda i: (i, 0)),
        jax.ShapeDtypeStruct((N, 128), jnp.float32),
    )(p2, ha, hb, dinv, W2, b2r, W3)

    p3 = prop128(ghat, src2, dst2, zeros128)

    b3r = b3.reshape(1, -1)
    wcp = jnp.pad(Wc, ((0, 0), (0, 128 - Wc.shape[1])))
    bcp = jnp.pad(bc, (0, 128 - bc.shape[0])).reshape(1, -1)
    z3, logit_pad = _tc_call(
        _k4_body, grid,
        [bspec((NC, _BN, 128), lambda i: (0, i, 0)),
         bspec((_BN, 128), lambda i: (i, 0)),
         bspec((_BN, 16), lambda i: (i, 0)),
         bspec((1, 128), lambda i: (0, 0)),
         bspec((128, 128), lambda i: (0, 0)),
         bspec((1, 128), lambda i: (0, 0))],
        [bspec((_BN, 128), lambda i: (i, 0)),
         bspec((_BN, 128), lambda i: (i, 0))],
        [jax.ShapeDtypeStruct((N, 128), jnp.float32),
         jax.ShapeDtypeStruct((N, 128), jnp.float32)],
    )(p3, ghat, dinv, b3r, wcp, bcp)

    n_link = pos_edge.shape[1] + neg_edge.shape[1]
    EP = -(-n_link // (NW * LB * 2)) * (NW * LB * 2)
    pad_idx = (jnp.arange(EP - n_link) % N).astype(jnp.int32)
    ia = jnp.concatenate([pos_edge[0].astype(jnp.int32),
                          neg_edge[0].astype(jnp.int32), pad_idx])
    ib = jnp.concatenate([pos_edge[1].astype(jnp.int32),
                          neg_edge[1].astype(jnp.int32), pad_idx])

    t_tab = _make_link(EP)(z3, ia, ib)

    t128 = t_tab.reshape(EP // 8, 128)
    gmat = (jnp.arange(128)[:, None] // 16
            == jnp.arange(8)[None, :]).astype(jnp.float32)
    bt = EP // 8 // 16
    preds8 = _tc_call(
        _k5_body, (16,),
        [bspec((bt, 128), lambda i: (i, 0)),
         bspec((128, 8), lambda i: (0, 0))],
        bspec((bt, 8), lambda i: (i, 0)),
        jax.ShapeDtypeStruct((EP // 8, 8), jnp.float32),
    )(t128, gmat)

    node_logits = logit_pad[:, :Wc.shape[1]]
    preds = preds8.reshape(EP)[:n_link]
    return z3, node_logits, preds


# speedup vs baseline: 1.0949x; 1.0949x over previous
import functools

import jax
import jax.numpy as jnp
from jax import lax
from jax.experimental import pallas as pl
from jax.experimental.pallas import tpu as pltpu
from jax.experimental.pallas import tpu_sc as plsc

NC = 2
NS = 16
NW = NC * NS

N = 10000
NPAD = 10240
ROWS_T = NPAD // NS
EB = 128
LB = 128

_mesh = plsc.VectorSubcoreMesh(
    core_axis_name="c", subcore_axis_name="s", num_cores=NC, num_subcores=NS)


def _make_deg(EPAD):
    ti = EPAD // NW // EB

    @functools.partial(
        pl.kernel,
        out_type=jax.ShapeDtypeStruct((NC, NPAD, 128), jnp.float32),
        mesh=_mesh,
        scratch_types=[
            pltpu.VMEM((ti, EB), jnp.int32),
            pltpu.VMEM((EB, 128), jnp.float32),
            pltpu.SemaphoreType.DMA,
            pltpu.VMEM_SHARED((NPAD, 128), jnp.float32),
        ],
    )
    def deg_k(dst2_hbm, zeros_hbm, ones_hbm, out_hbm, dall, ones_v, ssem,
              acc):
        c = lax.axis_index("c")
        s = lax.axis_index("s")
        w = s * NC + c
        pltpu.sync_copy(zeros_hbm, acc.at[pl.ds(s * ROWS_T, ROWS_T)])
        pltpu.sync_copy(ones_hbm, ones_v)
        pltpu.sync_copy(dst2_hbm.at[pl.ds(pl.multiple_of(w * ti, 8), ti)], dall)
        plsc.subcore_barrier()

        pltpu.async_copy(ones_v, acc.at[dall.at[0]], ssem, add=True)

        def step(j, carry):
            pltpu.async_copy(ones_v, acc.at[dall.at[j]], ssem, add=True)
            pltpu.make_async_copy(ones_v, acc.at[dall.at[0]], ssem).wait()
            return carry

        lax.fori_loop(1, ti, step, 0)
        pltpu.make_async_copy(ones_v, acc.at[dall.at[0]], ssem).wait()
        plsc.subcore_barrier()
        pltpu.sync_copy(acc.at[pl.ds(s * ROWS_T, ROWS_T)],
                        out_hbm.at[c, pl.ds(s * ROWS_T, ROWS_T)])

    return deg_k


def _make_prop(EPAD, split_cols):
    CH = 16
    ti = EPAD // (NS if split_cols else NW) // EB
    assert ti % CH == 0

    scratch = [
        pltpu.VMEM((CH, EB), jnp.int32),
        pltpu.VMEM((CH, EB), jnp.int32),
        pltpu.VMEM((EB, 128), jnp.float32),
        pltpu.VMEM((EB, 128), jnp.float32),
        pltpu.SemaphoreType.DMA,
        pltpu.SemaphoreType.DMA,
        pltpu.VMEM_SHARED((NPAD, 128), jnp.float32),
    ]

    def body(tabs, src2_hbm, dst2_hbm, zeros_hbm, out_hbm,
             sidx, didx, rows0, rows1, gsem, ssem, acc):
        c = lax.axis_index("c")
        s = lax.axis_index("s")
        pltpu.sync_copy(zeros_hbm, acc.at[pl.ds(s * ROWS_T, ROWS_T)])
        base_row = (s if split_cols else s * NC + c) * ti
        plsc.subcore_barrier()

        def run(tab):
            def g_issue(j, buf):
                pltpu.async_copy(tab.at[sidx.at[j]], buf, gsem)

            def g_wait(buf):
                pltpu.make_async_copy(tab.at[sidx.at[0]], buf, gsem).wait()

            def s_issue(j, buf):
                pltpu.async_copy(buf, acc.at[didx.at[j]], ssem, add=True)

            def s_wait():
                pltpu.make_async_copy(
                    rows0, acc.at[didx.at[0]], ssem).wait()

            def chunk(cc, carry):
                row0 = pl.multiple_of(base_row + cc * CH, 8)
                pltpu.sync_copy(src2_hbm.at[pl.ds(row0, CH)], sidx)
                pltpu.sync_copy(dst2_hbm.at[pl.ds(row0, CH)], didx)
                g_issue(0, rows0)
                g_wait(rows0)
                g_issue(1, rows1)
                s_issue(0, rows0)

                def step2(jj, c2):
                    k = 2 * jj
                    g_wait(rows1)
                    s_wait()
                    g_issue(k + 2, rows0)
                    s_issue(k + 1, rows1)
                    g_wait(rows0)
                    s_wait()
                    g_issue(k + 3, rows1)
                    s_issue(k + 2, rows0)
                    return c2

                lax.fori_loop(0, (CH - 2) // 2, step2, 0)
                g_wait(rows1)
                s_wait()
                s_issue(CH - 1, rows1)
                s_wait()
                return carry

            lax.fori_loop(0, ti // CH, chunk, 0)

        if split_cols:
            @pl.when(c == 0)
            def _():
                run(tabs[0])

            @pl.when(c == 1)
            def _():
                run(tabs[1])
        else:
            run(tabs[0])
        plsc.subcore_barrier()
        pltpu.sync_copy(acc.at[pl.ds(s * ROWS_T, ROWS_T)],
                        out_hbm.at[c, pl.ds(s * ROWS_T, ROWS_T)])

    kw = dict(
        out_type=jax.ShapeDtypeStruct((NC, NPAD, 128), jnp.float32),
        mesh=_mesh,
        scratch_types=scratch,
    )
    if split_cols:
        @functools.partial(pl.kernel, **kw)
        def prop_k(ta, tb, src_hbm, dst_hbm, zeros_hbm, out_hbm, *scr):
            body((ta, tb), src_hbm, dst_hbm, zeros_hbm, out_hbm, *scr)
    else:
        @functools.partial(pl.kernel, **kw)
        def prop_k(ta, src_hbm, dst_hbm, zeros_hbm, out_hbm, *scr):
            body((ta,), src_hbm, dst_hbm, zeros_hbm, out_hbm, *scr)
    return prop_k


def _make_link(EP):
    ti = EP // NW // LB
    assert ti % 2 == 0
    per_w = EP // NW

    @functools.partial(
        pl.kernel,
        out_type=jax.ShapeDtypeStruct((EP // 8, 128), jnp.float32),
        mesh=_mesh,
        scratch_types=[
            pltpu.VMEM((per_w,), jnp.int32),
            pltpu.VMEM((per_w,), jnp.int32),
            pltpu.VMEM((LB, 128), jnp.float32),
            pltpu.VMEM((LB, 128), jnp.float32),
            pltpu.VMEM((LB, 128), jnp.float32),
            pltpu.VMEM((LB, 128), jnp.float32),
            pltpu.VMEM((LB // 8, 128), jnp.float32),
            pltpu.SemaphoreType.DMA,
        ],
    )
    def link_k(tab_hbm, ia_hbm, ib_hbm, out_hbm,
               iav, ibv, ra0, rb0, ra1, rb1, tbuf, gsem):
        c = lax.axis_index("c")
        s = lax.axis_index("s")
        w = s * NC + c
        base = pl.multiple_of(w * per_w, 8)
        pltpu.sync_copy(ia_hbm.at[pl.ds(base, per_w)], iav)
        pltpu.sync_copy(ib_hbm.at[pl.ds(base, per_w)], ibv)

        def g_issue(j, bufa, bufb):
            off = pl.multiple_of(j * LB, 8)
            pltpu.async_copy(tab_hbm.at[iav.at[pl.ds(off, LB)]], bufa, gsem)
            pltpu.async_copy(tab_hbm.at[ibv.at[pl.ds(off, LB)]], bufb, gsem)

        def g_wait(bufa, bufb):
            pltpu.make_async_copy(
                tab_hbm.at[iav.at[pl.ds(0, LB)]], bufa, gsem).wait()
            pltpu.make_async_copy(
                tab_hbm.at[ibv.at[pl.ds(0, LB)]], bufb, gsem).wait()

        def compute(j, ra, rb):
            @plsc.parallel_loop(0, LB, step=1, unroll=4)
            def pair(p):
                t = ra[p, pl.ds(0, 16)] * rb[p, pl.ds(0, 16)]
                for k in range(1, 8):
                    t = t + (ra[p, pl.ds(16 * k, 16)]
                             * rb[p, pl.ds(16 * k, 16)])
                tbuf[p // 8, pl.ds((p % 8) * 16, 16)] = t

            pltpu.sync_copy(
                tbuf,
                out_hbm.at[pl.ds(pl.multiple_of((base + j * LB) // 8, 8), LB // 8)])

        g_issue(0, ra0, rb0)

        def step2(jj, carry):
            j0 = 2 * jj
            g_wait(ra0, rb0)
            g_issue(j0 + 1, ra1, rb1)
            compute(j0, ra0, rb0)
            g_wait(ra1, rb1)
            g_issue(lax.min(j0 + 2, ti - 1), ra0, rb0)
            compute(j0 + 1, ra1, rb1)
            return carry

        lax.fori_loop(0, ti // 2, step2, 0)
        g_wait(ra0, rb0)

    return link_k


_BN = 1000


def _tc_call(body, grid, in_specs, out_specs, out_shapes):
    return pl.pallas_call(
        body,
        grid=grid,
        in_specs=in_specs,
        out_specs=out_specs,
        out_shape=out_shapes,
    )


def _prep_body(x_ref, degp_ref, xhat_ref, dinv_ref):
    d = degp_ref[0] + degp_ref[1] + 1.0
    dv = lax.rsqrt(d)
    dinv_ref[...] = dv[:, :16]
    xhat_ref[...] = x_ref[...] * dv[:, 0:1]


def _k2_body(p_ref, xhat_ref, dinv_ref, w_ref, b_ref, ha_ref, hb_ref):
    dv = dinv_ref[:, 0:1]
    u = (p_ref[0] + p_ref[1] + xhat_ref[...]) * dv
    z = jnp.maximum(
        jnp.dot(u, w_ref[...], preferred_element_type=jnp.float32)
        + b_ref[...], 0.0)
    h = z * dv
    ha_ref[...] = h[:, :128]
    hb_ref[...] = h[:, 128:]


def _k3_body(p_ref, ha_ref, hb_ref, dinv_ref, w2_ref, b2_ref, w3_ref,
             g_ref):
    dv = dinv_ref[:, 0:1]
    ua = (p_ref[0] + ha_ref[...]) * dv
    ub = (p_ref[1] + hb_ref[...]) * dv
    u = jnp.concatenate([ua, ub], axis=1)
    z2 = jnp.maximum(
        jnp.dot(u, w2_ref[...], preferred_element_type=jnp.float32)
        + b2_ref[...], 0.0)
    g = jnp.dot(z2, w3_ref[...], preferred_element_type=jnp.float32)
    g_ref[...] = g * dv


def _k4_body(p_ref, g_ref, dinv_ref, b3_ref, wc_ref, bc_ref,
             z_ref, logit_ref):
    dv = dinv_ref[:, 0:1]
    z3 = jnp.maximum(
        (p_ref[0] + p_ref[1] + g_ref[...]) * dv + b3_ref[...], 0.0)
    z_ref[...] = z3
    logit_ref[...] = (
        jnp.dot(z3, wc_ref[...], preferred_element_type=jnp.float32)
        + bc_ref[...])


def _k5_body(t_ref, g_ref, out_ref):
    s = jnp.dot(t_ref[...], g_ref[...], preferred_element_type=jnp.float32)
    out_ref[...] = 1.0 / (1.0 + jnp.exp(-s))


def kernel(x, edge_index, pos_edge, neg_edge, W1, b1, W2, b2, W3, b3, Wc, bc):
    E = edge_index.shape[1]
    src = edge_index[0].astype(jnp.int32)
    dst = edge_index[1].astype(jnp.int32)

    EPAD = -(-E // (NW * 2 * EB)) * (NW * 2 * EB)
    pad_src = (jnp.arange(EPAD - E) % N).astype(jnp.int32)
    pad_dst = (N + jnp.arange(EPAD - E) % (NPAD - N)).astype(jnp.int32)
    src2 = jnp.concatenate([src, pad_src]).reshape(-1, EB)
    dst2 = jnp.concatenate([dst, pad_dst]).reshape(-1, EB)

    zeros128 = jnp.zeros((ROWS_T, 128), jnp.float32)
    ones128 = jnp.ones((EB, 128), jnp.float32)

    degp = _make_deg(EPAD)(dst2, zeros128, ones128)

    grid = (N // _BN,)
    bspec = pl.BlockSpec

    xhat, dinv = _tc_call(
        _prep_body, grid,
        [bspec((_BN, 128), lambda i: (i, 0)),
         bspec((NC, _BN, 128), lambda i: (0, i, 0))],
        [bspec((_BN, 128), lambda i: (i, 0)),
         bspec((_BN, 16), lambda i: (i, 0))],
        [jax.ShapeDtypeStruct((N, 128), jnp.float32),
         jax.ShapeDtypeStruct((N, 16), jnp.float32)],
    )(x, degp)

    prop128 = _make_prop(EPAD, split_cols=False)
    prop256 = _make_prop(EPAD, split_cols=True)

    p1 = prop128(xhat, src2, dst2, zeros128)

    b1r = b1.reshape(1, -1)
    ha, hb = _tc_call(
        _k2_body, grid,
        [bspec((NC, _BN, 128), lambda i: (0, i, 0)),
         bspec((_BN, 128), lambda i: (i, 0)),
         bspec((_BN, 16), lambda i: (i, 0)),
         bspec((128, 256), lambda i: (0, 0)),
         bspec((1, 256), lambda i: (0, 0))],
        [bspec((_BN, 128), lambda i: (i, 0)),
         bspec((_BN, 128), lambda i: (i, 0))],
        [jax.ShapeDtypeStruct((N, 128), jnp.float32),
         jax.ShapeDtypeStruct((N, 128), jnp.float32)],
    )(p1, xhat, dinv, W1, b1r)

    p2 = prop256(ha, hb, src2, dst2, zeros128)

    b2r = b2.reshape(1, -1)
    ghat = _tc_call(
        _k3_body, grid,
        [bspec((NC, _BN, 128), lambda i: (0, i, 0)),
         bspec((_BN, 128), lambda i: (i, 0)),
         bspec((_BN, 128), lambda i: (i, 0)),
         bspec((_BN, 16), lambda i: (i, 0)),
         bspec((256, 256), lambda i: (0, 0)),
         bspec((1, 256), lambda i: (0, 0)),
         bspec((256, 128), lambda i: (0, 0))],
        bspec((_BN, 128), lambda i: (i, 0)),
        jax.ShapeDtypeStruct((N, 128), jnp.float32),
    )(p2, ha, hb, dinv, W2, b2r, W3)

    p3 = prop128(ghat, src2, dst2, zeros128)

    b3r = b3.reshape(1, -1)
    wcp = jnp.pad(Wc, ((0, 0), (0, 128 - Wc.shape[1])))
    bcp = jnp.pad(bc, (0, 128 - bc.shape[0])).reshape(1, -1)
    z3, logit_pad = _tc_call(
        _k4_body, grid,
        [bspec((NC, _BN, 128), lambda i: (0, i, 0)),
         bspec((_BN, 128), lambda i: (i, 0)),
         bspec((_BN, 16), lambda i: (i, 0)),
         bspec((1, 128), lambda i: (0, 0)),
         bspec((128, 128), lambda i: (0, 0)),
         bspec((1, 128), lambda i: (0, 0))],
        [bspec((_BN, 128), lambda i: (i, 0)),
         bspec((_BN, 128), lambda i: (i, 0))],
        [jax.ShapeDtypeStruct((N, 128), jnp.float32),
         jax.ShapeDtypeStruct((N, 128), jnp.float32)],
    )(p3, ghat, dinv, b3r, wcp, bcp)

    n_link = pos_edge.shape[1] + neg_edge.shape[1]
    EP = -(-n_link // (NW * LB * 2)) * (NW * LB * 2)
    pad_idx = (jnp.arange(EP - n_link) % N).astype(jnp.int32)
    ia = jnp.concatenate([pos_edge[0].astype(jnp.int32),
                          neg_edge[0].astype(jnp.int32), pad_idx])
    ib = jnp.concatenate([pos_edge[1].astype(jnp.int32),
                          neg_edge[1].astype(jnp.int32), pad_idx])

    t128 = _make_link(EP)(z3, ia, ib)

    gmat = (jnp.arange(128)[:, None] // 16
            == jnp.arange(8)[None, :]).astype(jnp.float32)
    bt = EP // 8 // 16
    preds8 = _tc_call(
        _k5_body, (16,),
        [bspec((bt, 128), lambda i: (i, 0)),
         bspec((128, 8), lambda i: (0, 0))],
        bspec((bt, 8), lambda i: (i, 0)),
        jax.ShapeDtypeStruct((EP // 8, 8), jnp.float32),
    )(t128, gmat)

    node_logits = logit_pad[:, :Wc.shape[1]]
    preds = preds8.reshape(EP)[:n_link]
    return z3, node_logits, preds
